# baseline (speedup 1.0000x reference)
import functools

import jax
import jax.numpy as jnp
from jax import lax
from jax.experimental import pallas as pl
from jax.experimental.pallas import tpu as pltpu
from jax.experimental.pallas import tpu_sc as plsc

B, N, C = 2, 16384, 16
NPOINT = 1024
RADIUS_LIST = [0.1, 0.2, 0.4]
NSAMPLE_LIST = [16, 32, 64]
NROW = 128
NS_TOTAL = sum(NSAMPLE_LIST)
IDX_TOTAL = B * NPOINT * NS_TOTAL
NWORK = 32
ROWS_PER_W = IDX_TOTAL // NWORK
GCHUNK = 128
NCHUNK = ROWS_PER_W // GCHUNK
DPAD = 32


def _fps_body(xt_ref, xs_ref, idx_ref, nxyz_ref, dist_ref):
    row = lax.broadcasted_iota(jnp.int32, (NROW, 128), 0)
    col = lax.broadcasted_iota(jnp.int32, (NROW, 128), 1)
    iota = row * 128 + col
    dist_ref[...] = jnp.full((B, NROW, 128), 1e10, dtype=jnp.float32)

    def body(i, fars):
        ds = []
        for b in range(B):
            far = fars[b]
            idx_ref[b, i] = far
            x = xt_ref[b, 0]
            y = xt_ref[b, 1]
            z = xt_ref[b, 2]
            cx = xs_ref[b, 0, far]
            cy = xs_ref[b, 1, far]
            cz = xs_ref[b, 2, far]
            nxyz_ref[0, b, i] = cx
            nxyz_ref[1, b, i] = cy
            nxyz_ref[2, b, i] = cz
            dx = x - cx
            dy = y - cy
            dz = z - cz
            ds.append(dx * dx + dy * dy + dz * dz)
        dmin = jnp.minimum(dist_ref[...], jnp.stack(ds))
        dist_ref[...] = dmin
        m = jnp.max(dmin, axis=(1, 2), keepdims=True)
        key = jnp.where(dmin == m, iota[None], jnp.int32(2**31 - 1))
        far_n = jnp.min(key, axis=(1, 2))
        return tuple(far_n[b] for b in range(B))

    lax.fori_loop(0, NPOINT, body, tuple(jnp.int32(0) for _ in range(B)))


def _fps(xyz):
    xyzT = xyz.transpose(0, 2, 1)
    xt = xyzT.reshape(B, 3, NROW, 128)
    fps_idx, new_xyz = pl.pallas_call(
        _fps_body,
        out_shape=[
            jax.ShapeDtypeStruct((B, NPOINT), jnp.int32),
            jax.ShapeDtypeStruct((3, B, NPOINT), jnp.float32),
        ],
        in_specs=[pl.BlockSpec(memory_space=pltpu.VMEM),
                  pl.BlockSpec(memory_space=pltpu.SMEM)],
        out_specs=[
            pl.BlockSpec(memory_space=pltpu.SMEM),
            pl.BlockSpec(memory_space=pltpu.SMEM),
        ],
        scratch_shapes=[pltpu.VMEM((B, NROW, 128), jnp.float32)],
    )(xt, xyzT)
    return fps_idx, new_xyz.transpose(1, 2, 0)


def _sc_gather(xpad, idx_flat):
    mesh = plsc.VectorSubcoreMesh(core_axis_name="c", subcore_axis_name="s")

    @functools.partial(
        pl.kernel,
        mesh=mesh,
        out_type=jax.ShapeDtypeStruct((IDX_TOTAL, DPAD), jnp.float32),
        scratch_types=[
            pltpu.VMEM((2 * GCHUNK,), jnp.int32),
            pltpu.VMEM((2 * GCHUNK, DPAD), jnp.float32),
            pltpu.SemaphoreType.DMA,
        ],
        compiler_params=pltpu.CompilerParams(
            use_tc_tiling_on_sc=False, needs_layout_passes=False),
    )
    def gk(xpad_hbm, idx_hbm, out_hbm, idx_v, rows_v, sem):
        wid = lax.axis_index("s") * 2 + lax.axis_index("c")
        base = wid * ROWS_PER_W
        pltpu.sync_copy(idx_hbm.at[pl.ds(base, GCHUNK)],
                        idx_v.at[pl.ds(0, GCHUNK)])
        pltpu.async_copy(xpad_hbm.at[idx_v.at[pl.ds(0, GCHUNK)]],
                         rows_v.at[pl.ds(0, GCHUNK)], sem)

        def chunk(t, carry):
            p = (t % 2) * GCHUNK
            pn = ((t + 1) % 2) * GCHUNK
            off = base + t * GCHUNK
            pltpu.make_async_copy(xpad_hbm.at[pl.ds(0, GCHUNK)],
                                  rows_v.at[pl.ds(0, GCHUNK)], sem).wait()
            noff = base + jnp.minimum(t + 1, NCHUNK - 1) * GCHUNK
            pltpu.sync_copy(idx_hbm.at[pl.ds(noff, GCHUNK)],
                            idx_v.at[pl.ds(pn, GCHUNK)])
            pltpu.async_copy(xpad_hbm.at[idx_v.at[pl.ds(pn, GCHUNK)]],
                             rows_v.at[pl.ds(pn, GCHUNK)], sem)
            pltpu.sync_copy(rows_v.at[pl.ds(p, GCHUNK)],
                            out_hbm.at[pl.ds(off, GCHUNK)])
            return carry

        lax.fori_loop(0, NCHUNK, chunk, jnp.int32(0))
        pltpu.make_async_copy(xpad_hbm.at[pl.ds(0, GCHUNK)],
                              rows_v.at[pl.ds(0, GCHUNK)], sem).wait()

    return gk(xpad, idx_flat)


GD = 128


def _sqd_body(nx_ref, xt_ref, d_ref):
    nx = nx_ref[0]
    xt = xt_ref[0]
    aa = jnp.sum(nx * nx, axis=1, keepdims=True)
    bb = (xt[0] * xt[0] + xt[1] * xt[1] + xt[2] * xt[2])[None, :]
    ab = jnp.dot(nx, xt, preferred_element_type=jnp.float32)
    d_ref[0] = aa + bb - 2.0 * ab


def _sqdists(new_xyz, xyzT):
    return pl.pallas_call(
        _sqd_body,
        grid=(B, NPOINT // GD),
        out_shape=jax.ShapeDtypeStruct((B, NPOINT, N), jnp.float32),
        in_specs=[
            pl.BlockSpec((1, GD, 3), lambda b, j: (b, j, 0)),
            pl.BlockSpec((1, 3, N), lambda b, j: (b, 0, 0)),
        ],
        out_specs=pl.BlockSpec((1, GD, N), lambda b, j: (b, j, 0)),
    )(new_xyz, xyzT)


CPW = B * NPOINT // NWORK
STRIDES = [ns + 32 for ns in NSAMPLE_LIST]
NCHK = N // 16


def _sc_ballquery(d_flat):
    mesh = plsc.VectorSubcoreMesh(core_axis_name="c", subcore_axis_name="s")
    r2s = [r * r for r in RADIUS_LIST]

    @functools.partial(
        pl.kernel,
        mesh=mesh,
        out_type=[
            jax.ShapeDtypeStruct((B * NPOINT * ns,), jnp.int32)
            for ns in NSAMPLE_LIST
        ],
        scratch_types=[
            pltpu.VMEM((3 * N,), jnp.float32),
            pltpu.VMEM((CPW * STRIDES[0] + 16,), jnp.int32),
            pltpu.VMEM((CPW * STRIDES[1] + 16,), jnp.int32),
            pltpu.VMEM((CPW * STRIDES[2] + 16,), jnp.int32),
            pltpu.SemaphoreType.DMA,
        ],
        compiler_params=pltpu.CompilerParams(
            use_tc_tiling_on_sc=False, needs_layout_passes=False),
    )
    def bq(d_hbm, o0_hbm, o1_hbm, o2_hbm, dbuf, buf0, buf1, buf2, sem):
        wid = lax.axis_index("s") * 2 + lax.axis_index("c")
        b = wid // 16
        k16 = wid % 16
        base_pt = b * N
        base_row = b * NPOINT + k16
        lane = lax.broadcasted_iota(jnp.int32, (16,), 0)
        bufs = (buf0, buf1, buf2)
        outs = (o0_hbm, o1_hbm, o2_hbm)
        pltpu.async_copy(d_hbm.at[pl.ds(base_row * N, N)],
                         dbuf.at[pl.ds(0, N)], sem)
        pltpu.async_copy(d_hbm.at[pl.ds((base_row + 16) * N, N)],
                         dbuf.at[pl.ds(N, N)], sem)

        def per_centroid(ci, carry):
            grow = base_row + ci * 16
            pltpu.make_async_copy(d_hbm.at[pl.ds(0, N)],
                                  dbuf.at[pl.ds(0, N)], sem).wait()
            nxt = jnp.minimum(ci + 2, CPW - 1)
            pltpu.async_copy(
                d_hbm.at[pl.ds((base_row + nxt * 16) * N, N)],
                dbuf.at[pl.ds(((ci + 2) % 3) * N, N)], sem)
            pbase = (ci % 3) * N

            for r in range(3):
                ns = NSAMPLE_LIST[r]
                stride = STRIDES[r]
                pos0 = ci * stride
                trash = jnp.int32(CPW * stride)
                r2 = r2s[r]

                def cond(st):
                    t, o = st
                    return (t < NCHK) & (o < ns)

                def body(st):
                    t, o = st
                    off = t * 16
                    da = dbuf[pl.ds(pbase + off, 16)]
                    db = dbuf[pl.ds(pbase + off + 16, 16)]
                    ma = da <= r2
                    mb = db <= r2
                    cnta = plsc.all_reduce_population_count(ma)[0]
                    cntb = plsc.all_reduce_population_count(mb)[0]
                    pca = plsc.cumsum(ma.astype(jnp.int32))
                    pcb = plsc.cumsum(mb.astype(jnp.int32))
                    ta = jnp.where(ma, pos0 + o + (pca - 1), trash)
                    plsc.store_scatter(bufs[r], [ta], lane + (off + base_pt))
                    tb = jnp.where(
                        mb, pos0 + (o + cnta) + (pcb - 1), trash)
                    plsc.store_scatter(bufs[r], [tb],
                                       lane + (off + 16 + base_pt))
                    return (t + 2, o + (cnta + cntb))

                z32 = jnp.int32(0)
                _, o = lax.while_loop(cond, body, (z32, z32))

                cnt = jnp.minimum(o, ns)
                first = bufs[r][pl.ds(pos0, 16)][0]
                for k in range(ns // 16):
                    sl = pl.ds(pos0 + k * 16, 16)
                    v = bufs[r][sl]
                    vfix = jnp.where(lane + (k * 16) < cnt, v, first)
                    bufs[r][sl] = vfix
                pltpu.sync_copy(
                    bufs[r].at[pl.ds(pos0, ns)],
                    outs[r].at[pl.ds(grow * ns, ns)])
            return carry

        lax.fori_loop(0, CPW, per_centroid, jnp.int32(0))
        pltpu.make_async_copy(d_hbm.at[pl.ds(0, N)],
                              dbuf.at[pl.ds(0, N)], sem).wait()
        pltpu.make_async_copy(d_hbm.at[pl.ds(0, N)],
                              dbuf.at[pl.ds(0, N)], sem).wait()

    return bq(d_flat)


def _mlp_body(xg_ref, cpad_ref, w1_ref, b1_ref, w2_ref, b2_ref, out_ref):
    gm, ns, dpad = xg_ref.shape
    x = xg_ref[...] - cpad_ref[...][:, None, :]
    x2 = x.reshape(gm * ns, dpad)
    h = jnp.maximum(jnp.dot(x2, w1_ref[...],
                            preferred_element_type=jnp.float32)
                    + b1_ref[...], 0.0)
    h2 = jnp.maximum(jnp.dot(h, w2_ref[...],
                             preferred_element_type=jnp.float32)
                     + b2_ref[...], 0.0)
    out_ref[...] = jnp.max(h2.reshape(gm, ns, 64), axis=1)


def _mlp_max(xg_s, cpad, w1pad, b1, w2, b2, ns):
    gm = 8192 // ns
    grid = (B * NPOINT // gm,)
    return pl.pallas_call(
        _mlp_body,
        grid=grid,
        out_shape=jax.ShapeDtypeStruct((B * NPOINT, 64), jnp.float32),
        in_specs=[
            pl.BlockSpec((gm, ns, DPAD), lambda j: (j, 0, 0)),
            pl.BlockSpec((gm, DPAD), lambda j: (j, 0)),
            pl.BlockSpec((DPAD, 32), lambda j: (0, 0)),
            pl.BlockSpec((1, 32), lambda j: (0, 0)),
            pl.BlockSpec((32, 64), lambda j: (0, 0)),
            pl.BlockSpec((1, 64), lambda j: (0, 0)),
        ],
        out_specs=pl.BlockSpec((gm, 64), lambda j: (j, 0)),
    )(xg_s, cpad, w1pad, b1.reshape(1, 32), w2, b2.reshape(1, 64))


def kernel(xyz, points, W0_0, b0_0, W0_1, b0_1, W1_0, b1_0, W1_1, b1_1,
           W2_0, b2_0, W2_1, b2_1):
    params = [[(W0_0, b0_0), (W0_1, b0_1)], [(W1_0, b1_0), (W1_1, b1_1)],
              [(W2_0, b2_0), (W2_1, b2_1)]]
    fps_idx, new_xyz = _fps(xyz)

    zcols = jnp.zeros((B * N, DPAD - C - 3), jnp.float32)
    xpad = jnp.concatenate(
        [points.reshape(B * N, C), xyz.reshape(B * N, 3), zcols], axis=1)
    czero = jnp.zeros((B * NPOINT, C), jnp.float32)
    cpad = jnp.concatenate(
        [czero, new_xyz.reshape(B * NPOINT, 3),
         jnp.zeros((B * NPOINT, DPAD - C - 3), jnp.float32)], axis=1)

    sq = _sqdists(new_xyz, xyz.transpose(0, 2, 1))
    idx_parts = _sc_ballquery(sq.reshape(-1))
    idx_flat = jnp.concatenate(idx_parts)

    xg = _sc_gather(xpad, idx_flat)

    outs = []
    off = 0
    for i in range(3):
        ns = NSAMPLE_LIST[i]
        rows = B * NPOINT * ns
        xs = xg[off:off + rows].reshape(B * NPOINT, ns, DPAD)
        off += rows
        (W1, b1), (W2, b2) = params[i]
        w1pad = jnp.concatenate(
            [W1, jnp.zeros((DPAD - C - 3, W1.shape[1]), jnp.float32)], axis=0)
        o = _mlp_max(xs, cpad, w1pad, b1, W2, b2, ns)
        outs.append(o.reshape(B, NPOINT, 64))
    new_points_concat = jnp.concatenate(outs, axis=-1)
    return (new_xyz, new_points_concat)


# speedup vs baseline: 1.1331x; 1.1331x over previous
import functools

import jax
import jax.numpy as jnp
from jax import lax
from jax.experimental import pallas as pl
from jax.experimental.pallas import tpu as pltpu
from jax.experimental.pallas import tpu_sc as plsc

B, N, C = 2, 16384, 16
NPOINT = 1024
RADIUS_LIST = [0.1, 0.2, 0.4]
NSAMPLE_LIST = [16, 32, 64]
NROW = 128
NS_TOTAL = sum(NSAMPLE_LIST)
IDX_TOTAL = B * NPOINT * NS_TOTAL
NWORK = 32
ROWS_PER_W = IDX_TOTAL // NWORK
GCHUNK = 128
NCHUNK = ROWS_PER_W // GCHUNK
DPAD = 32


def _fps_body(xt_ref, xs_ref, idx_ref, nxyz_ref, dist_ref):
    row = lax.broadcasted_iota(jnp.int32, (NROW, 128), 0)
    col = lax.broadcasted_iota(jnp.int32, (NROW, 128), 1)
    iota = row * 128 + col
    dist_ref[...] = jnp.full((B, NROW, 128), 1e10, dtype=jnp.float32)

    def body(i, fars):
        ds = []
        for b in range(B):
            far = fars[b]
            idx_ref[b, i] = far
            x = xt_ref[b, 0]
            y = xt_ref[b, 1]
            z = xt_ref[b, 2]
            cx = xs_ref[b, 0, far]
            cy = xs_ref[b, 1, far]
            cz = xs_ref[b, 2, far]
            nxyz_ref[0, b, i] = cx
            nxyz_ref[1, b, i] = cy
            nxyz_ref[2, b, i] = cz
            dx = x - cx
            dy = y - cy
            dz = z - cz
            ds.append(dx * dx + dy * dy + dz * dz)
        dmin = jnp.minimum(dist_ref[...], jnp.stack(ds))
        dist_ref[...] = dmin
        m = jnp.max(dmin, axis=(1, 2), keepdims=True)
        key = jnp.where(dmin == m, iota[None], jnp.int32(2**31 - 1))
        far_n = jnp.min(key, axis=(1, 2))
        return tuple(far_n[b] for b in range(B))

    lax.fori_loop(0, NPOINT, body, tuple(jnp.int32(0) for _ in range(B)))


def _fps(xyz):
    xyzT = xyz.transpose(0, 2, 1)
    xt = xyzT.reshape(B, 3, NROW, 128)
    fps_idx, new_xyz = pl.pallas_call(
        _fps_body,
        out_shape=[
            jax.ShapeDtypeStruct((B, NPOINT), jnp.int32),
            jax.ShapeDtypeStruct((3, B, NPOINT), jnp.float32),
        ],
        in_specs=[pl.BlockSpec(memory_space=pltpu.VMEM),
                  pl.BlockSpec(memory_space=pltpu.SMEM)],
        out_specs=[
            pl.BlockSpec(memory_space=pltpu.SMEM),
            pl.BlockSpec(memory_space=pltpu.SMEM),
        ],
        scratch_shapes=[pltpu.VMEM((B, NROW, 128), jnp.float32)],
    )(xt, xyzT)
    return fps_idx, new_xyz.transpose(1, 2, 0)


def _sc_gather(xpad, idx_flat):
    mesh = plsc.VectorSubcoreMesh(core_axis_name="c", subcore_axis_name="s")

    @functools.partial(
        pl.kernel,
        mesh=mesh,
        out_type=jax.ShapeDtypeStruct((IDX_TOTAL, DPAD), jnp.float32),
        scratch_types=[
            pltpu.VMEM((2 * GCHUNK,), jnp.int32),
            pltpu.VMEM((2 * GCHUNK, DPAD), jnp.float32),
            pltpu.SemaphoreType.DMA,
        ],
        compiler_params=pltpu.CompilerParams(
            use_tc_tiling_on_sc=False, needs_layout_passes=False),
    )
    def gk(xpad_hbm, idx_hbm, out_hbm, idx_v, rows_v, sem):
        wid = lax.axis_index("s") * 2 + lax.axis_index("c")
        base = wid * ROWS_PER_W
        pltpu.sync_copy(idx_hbm.at[pl.ds(base, GCHUNK)],
                        idx_v.at[pl.ds(0, GCHUNK)])
        pltpu.async_copy(xpad_hbm.at[idx_v.at[pl.ds(0, GCHUNK)]],
                         rows_v.at[pl.ds(0, GCHUNK)], sem)

        def chunk(t, carry):
            p = (t % 2) * GCHUNK
            pn = ((t + 1) % 2) * GCHUNK
            off = base + t * GCHUNK
            pltpu.make_async_copy(xpad_hbm.at[pl.ds(0, GCHUNK)],
                                  rows_v.at[pl.ds(0, GCHUNK)], sem).wait()
            noff = base + jnp.minimum(t + 1, NCHUNK - 1) * GCHUNK
            pltpu.sync_copy(idx_hbm.at[pl.ds(noff, GCHUNK)],
                            idx_v.at[pl.ds(pn, GCHUNK)])
            pltpu.async_copy(xpad_hbm.at[idx_v.at[pl.ds(pn, GCHUNK)]],
                             rows_v.at[pl.ds(pn, GCHUNK)], sem)
            pltpu.sync_copy(rows_v.at[pl.ds(p, GCHUNK)],
                            out_hbm.at[pl.ds(off, GCHUNK)])
            return carry

        lax.fori_loop(0, NCHUNK, chunk, jnp.int32(0))
        pltpu.make_async_copy(xpad_hbm.at[pl.ds(0, GCHUNK)],
                              rows_v.at[pl.ds(0, GCHUNK)], sem).wait()

    return gk(xpad, idx_flat)


GD = 128


def _sqd_body(nx_ref, xt_ref, d_ref):
    nx = nx_ref[0]
    xt = xt_ref[0]
    aa = jnp.sum(nx * nx, axis=1, keepdims=True)
    bb = (xt[0] * xt[0] + xt[1] * xt[1] + xt[2] * xt[2])[None, :]
    ab = jnp.dot(nx, xt, preferred_element_type=jnp.float32)
    d_ref[0] = aa + bb - 2.0 * ab


def _sqdists(new_xyz, xyzT):
    return pl.pallas_call(
        _sqd_body,
        grid=(B, NPOINT // GD),
        out_shape=jax.ShapeDtypeStruct((B, NPOINT, N), jnp.float32),
        in_specs=[
            pl.BlockSpec((1, GD, 3), lambda b, j: (b, j, 0)),
            pl.BlockSpec((1, 3, N), lambda b, j: (b, 0, 0)),
        ],
        out_specs=pl.BlockSpec((1, GD, N), lambda b, j: (b, j, 0)),
    )(new_xyz, xyzT)


CPW = B * NPOINT // NWORK
STRIDES = [ns + 64 for ns in NSAMPLE_LIST]
NCHK = N // 16


def _sc_ballquery(d_flat):
    mesh = plsc.VectorSubcoreMesh(core_axis_name="c", subcore_axis_name="s")
    r2s = [r * r for r in RADIUS_LIST]

    @functools.partial(
        pl.kernel,
        mesh=mesh,
        out_type=[
            jax.ShapeDtypeStruct((B * NPOINT * ns,), jnp.int32)
            for ns in NSAMPLE_LIST
        ],
        scratch_types=[
            pltpu.VMEM((3 * N,), jnp.float32),
            pltpu.VMEM((CPW * STRIDES[0] + 16,), jnp.int32),
            pltpu.VMEM((CPW * STRIDES[1] + 16,), jnp.int32),
            pltpu.VMEM((CPW * STRIDES[2] + 16,), jnp.int32),
            pltpu.SemaphoreType.DMA,
        ],
        compiler_params=pltpu.CompilerParams(
            use_tc_tiling_on_sc=False, needs_layout_passes=False),
    )
    def bq(d_hbm, o0_hbm, o1_hbm, o2_hbm, dbuf, buf0, buf1, buf2, sem):
        wid = lax.axis_index("s") * 2 + lax.axis_index("c")
        b = wid // 16
        k16 = wid % 16
        base_pt = b * N
        base_row = b * NPOINT + k16
        lane = lax.broadcasted_iota(jnp.int32, (16,), 0)
        bufs = (buf0, buf1, buf2)
        outs = (o0_hbm, o1_hbm, o2_hbm)
        pltpu.async_copy(d_hbm.at[pl.ds(base_row * N, N)],
                         dbuf.at[pl.ds(0, N)], sem)
        pltpu.async_copy(d_hbm.at[pl.ds((base_row + 16) * N, N)],
                         dbuf.at[pl.ds(N, N)], sem)

        def per_centroid(ci, carry):
            grow = base_row + ci * 16
            pltpu.make_async_copy(d_hbm.at[pl.ds(0, N)],
                                  dbuf.at[pl.ds(0, N)], sem).wait()
            nxt = jnp.minimum(ci + 2, CPW - 1)
            pltpu.async_copy(
                d_hbm.at[pl.ds((base_row + nxt * 16) * N, N)],
                dbuf.at[pl.ds(((ci + 2) % 3) * N, N)], sem)
            pbase = (ci % 3) * N

            for r in range(3):
                ns = NSAMPLE_LIST[r]
                stride = STRIDES[r]
                pos0 = ci * stride
                trash = jnp.int32(CPW * stride)
                r2 = r2s[r]

                def cond(st):
                    t, o = st
                    return (t < NCHK) & (o < ns)

                def body(st):
                    t, o = st
                    off = t * 16
                    ms, cnts = [], []
                    for u in range(4):
                        d = dbuf[pl.ds(pbase + off + 16 * u, 16)]
                        m = d <= r2
                        ms.append(m)
                        cnts.append(plsc.all_reduce_population_count(m)[0])
                    ob = o
                    for u in range(4):
                        pc = plsc.cumsum(ms[u].astype(jnp.int32))
                        tg = jnp.where(ms[u], pos0 + ob + (pc - 1), trash)
                        plsc.store_scatter(bufs[r], [tg],
                                           lane + (off + 16 * u + base_pt))
                        ob = ob + cnts[u]
                    return (t + 4, ob)

                z32 = jnp.int32(0)
                _, o = lax.while_loop(cond, body, (z32, z32))

                cnt = jnp.minimum(o, ns)
                first = bufs[r][pl.ds(pos0, 16)][0]
                for k in range(ns // 16):
                    sl = pl.ds(pos0 + k * 16, 16)
                    v = bufs[r][sl]
                    vfix = jnp.where(lane + (k * 16) < cnt, v, first)
                    bufs[r][sl] = vfix
                pltpu.sync_copy(
                    bufs[r].at[pl.ds(pos0, ns)],
                    outs[r].at[pl.ds(grow * ns, ns)])
            return carry

        lax.fori_loop(0, CPW, per_centroid, jnp.int32(0))
        pltpu.make_async_copy(d_hbm.at[pl.ds(0, N)],
                              dbuf.at[pl.ds(0, N)], sem).wait()
        pltpu.make_async_copy(d_hbm.at[pl.ds(0, N)],
                              dbuf.at[pl.ds(0, N)], sem).wait()

    return bq(d_flat)


def _mlp_body(xg_ref, cpad_ref, w1_ref, b1_ref, w2_ref, b2_ref, out_ref):
    gm, ns, dpad = xg_ref.shape
    x = xg_ref[...] - cpad_ref[...][:, None, :]
    x2 = x.reshape(gm * ns, dpad)
    h = jnp.maximum(jnp.dot(x2, w1_ref[...],
                            preferred_element_type=jnp.float32)
                    + b1_ref[...], 0.0)
    h2 = jnp.maximum(jnp.dot(h, w2_ref[...],
                             preferred_element_type=jnp.float32)
                     + b2_ref[...], 0.0)
    out_ref[...] = jnp.max(h2.reshape(gm, ns, 64), axis=1)


def _mlp_max(xg_s, cpad, w1pad, b1, w2, b2, ns):
    gm = 8192 // ns
    grid = (B * NPOINT // gm,)
    return pl.pallas_call(
        _mlp_body,
        grid=grid,
        out_shape=jax.ShapeDtypeStruct((B * NPOINT, 64), jnp.float32),
        in_specs=[
            pl.BlockSpec((gm, ns, DPAD), lambda j: (j, 0, 0)),
            pl.BlockSpec((gm, DPAD), lambda j: (j, 0)),
            pl.BlockSpec((DPAD, 32), lambda j: (0, 0)),
            pl.BlockSpec((1, 32), lambda j: (0, 0)),
            pl.BlockSpec((32, 64), lambda j: (0, 0)),
            pl.BlockSpec((1, 64), lambda j: (0, 0)),
        ],
        out_specs=pl.BlockSpec((gm, 64), lambda j: (j, 0)),
    )(xg_s, cpad, w1pad, b1.reshape(1, 32), w2, b2.reshape(1, 64))


def kernel(xyz, points, W0_0, b0_0, W0_1, b0_1, W1_0, b1_0, W1_1, b1_1,
           W2_0, b2_0, W2_1, b2_1):
    params = [[(W0_0, b0_0), (W0_1, b0_1)], [(W1_0, b1_0), (W1_1, b1_1)],
              [(W2_0, b2_0), (W2_1, b2_1)]]
    fps_idx, new_xyz = _fps(xyz)

    zcols = jnp.zeros((B * N, DPAD - C - 3), jnp.float32)
    xpad = jnp.concatenate(
        [points.reshape(B * N, C), xyz.reshape(B * N, 3), zcols], axis=1)
    czero = jnp.zeros((B * NPOINT, C), jnp.float32)
    cpad = jnp.concatenate(
        [czero, new_xyz.reshape(B * NPOINT, 3),
         jnp.zeros((B * NPOINT, DPAD - C - 3), jnp.float32)], axis=1)

    sq = _sqdists(new_xyz, xyz.transpose(0, 2, 1))
    idx_parts = _sc_ballquery(sq.reshape(-1))
    idx_flat = jnp.concatenate(idx_parts)

    xg = _sc_gather(xpad, idx_flat)

    outs = []
    off = 0
    for i in range(3):
        ns = NSAMPLE_LIST[i]
        rows = B * NPOINT * ns
        xs = xg[off:off + rows].reshape(B * NPOINT, ns, DPAD)
        off += rows
        (W1, b1), (W2, b2) = params[i]
        w1pad = jnp.concatenate(
            [W1, jnp.zeros((DPAD - C - 3, W1.shape[1]), jnp.float32)], axis=0)
        o = _mlp_max(xs, cpad, w1pad, b1, W2, b2, ns)
        outs.append(o.reshape(B, NPOINT, 64))
    new_points_concat = jnp.concatenate(outs, axis=-1)
    return (new_xyz, new_points_concat)
